# baseline (speedup 1.0000x reference)
import jax
import jax.numpy as jnp
from jax.experimental import pallas as pl

NUM_EXPERTS = 16
HIDDEN = 64
FLAT = NUM_EXPERTS * HIDDEN
GPAD = 128


def _moe_body(x_ref, wcat_ref, bcat_ref, gw2t_ref, gb2_ref,
              w2_ref, eb2_ref, emat_ref, out_ref):
    xb16 = x_ref[...]
    f = xb16.shape[0]

    ha = jnp.maximum(
        jnp.dot(xb16, wcat_ref[...], preferred_element_type=jnp.float32)
        + bcat_ref[...], 0.0)
    h = ha[:, :FLAT]
    g1 = ha[:, FLAT:]

    logits = (jnp.dot(g1, gw2t_ref[...], preferred_element_type=jnp.float32)
              + gb2_ref[...])
    m = jnp.max(logits, axis=1, keepdims=True)
    el = jnp.exp(logits - m)
    z = jnp.sum(el, axis=1, keepdims=True)

    m1 = jnp.max(el, axis=1, keepdims=True)
    el2 = jnp.where(el == m1, -1.0, el)
    m2 = jnp.max(el2, axis=1, keepdims=True)
    w = jnp.where(el >= m2, el, 0.0) / z

    c = jax.lax.dot_general(w, h, (((0,), (0,)), ((), ())),
                            preferred_element_type=jnp.float32)
    s = jnp.sum(c * emat_ref[...], axis=0, keepdims=True)
    wsum = jnp.sum(w, axis=0, keepdims=True)

    out = (jnp.dot(s.astype(jnp.bfloat16), w2_ref[...],
                   preferred_element_type=jnp.float32)
           + jnp.dot(wsum, eb2_ref[...], preferred_element_type=jnp.float32))
    out_ref[...] = (out * (1.0 / f))[None]


def kernel(x, gw1, gb1, gw2, gb2, ew1, eb1, ew2, eb2):
    B, F, D = x.shape
    E, H, _ = ew1.shape
    O = ew2.shape[1]

    xf = x.reshape(B * F, D).astype(jnp.bfloat16)
    w1t = ew1.reshape(E * H, D).T.astype(jnp.bfloat16)
    gw1t = gw1.T.astype(jnp.bfloat16)
    wcat = jnp.concatenate(
        [w1t, gw1t, jnp.zeros((D, GPAD - H), jnp.bfloat16)], axis=1)
    bcat = jnp.concatenate(
        [eb1.reshape(1, E * H), gb1.reshape(1, H),
         jnp.zeros((1, GPAD - H), jnp.float32)], axis=1)
    gw2tp = jnp.concatenate(
        [gw2.T, jnp.zeros((GPAD - H, E), jnp.float32)], axis=0)
    gb2r = gb2.reshape(1, E)
    w2 = ew2.transpose(0, 2, 1).reshape(E * H, O).astype(jnp.bfloat16)
    emat = jnp.kron(jnp.eye(E, dtype=x.dtype), jnp.ones((1, H), dtype=x.dtype))

    full = lambda *shape: pl.BlockSpec(shape, lambda b: (0,) * len(shape))
    out = pl.pallas_call(
        _moe_body,
        grid=(B,),
        in_specs=[
            pl.BlockSpec((F, D), lambda b: (b, 0)),
            full(D, FLAT + GPAD), full(1, FLAT + GPAD),
            full(GPAD, E), full(1, E),
            full(E * H, O), full(E, O),
            full(E, E * H),
        ],
        out_specs=pl.BlockSpec((1, 1, O), lambda b: (b, 0, 0)),
        out_shape=jax.ShapeDtypeStruct((B, 1, O), x.dtype),
    )(xf, wcat, bcat, gw2tp, gb2r, w2, eb2, emat)
    return out.reshape(B, 1, 1, O)


# speedup vs baseline: 1.3363x; 1.3363x over previous
import jax
import jax.numpy as jnp
from jax.experimental import pallas as pl

NUM_EXPERTS = 16
HIDDEN = 64
FLAT = NUM_EXPERTS * HIDDEN
GPAD = 128


def _moe_body(x_ref, wcat_ref, bcat_ref, gw2t_ref, gb2_ref,
              w2_ref, eb2_ref, emat_ref, out_ref):
    xb16 = x_ref[...].astype(jnp.bfloat16)
    f = xb16.shape[0]

    ha = jnp.maximum(
        jnp.dot(xb16, wcat_ref[...], preferred_element_type=jnp.float32)
        + bcat_ref[...], 0.0)
    h = ha[:, :FLAT]
    g1 = ha[:, FLAT:]

    logits = (jnp.dot(g1, gw2t_ref[...], preferred_element_type=jnp.float32)
              + gb2_ref[...])
    m = jnp.max(logits, axis=1, keepdims=True)
    el = jnp.exp(logits - m)
    z = jnp.sum(el, axis=1, keepdims=True)

    m1 = jnp.max(el, axis=1, keepdims=True)
    el2 = jnp.where(el == m1, -1.0, el)
    m2 = jnp.max(el2, axis=1, keepdims=True)
    w = jnp.where(el >= m2, el, 0.0) / z

    c = jax.lax.dot_general(w, h, (((0,), (0,)), ((), ())),
                            preferred_element_type=jnp.float32)
    s = jnp.sum(c * emat_ref[...], axis=0, keepdims=True)
    wsum = jnp.sum(w, axis=0, keepdims=True)

    out = (jnp.dot(s.astype(jnp.bfloat16), w2_ref[...],
                   preferred_element_type=jnp.float32)
           + jnp.dot(wsum, eb2_ref[...], preferred_element_type=jnp.float32))
    out_ref[...] = (out * (1.0 / f))[None]


def kernel(x, gw1, gb1, gw2, gb2, ew1, eb1, ew2, eb2):
    B, F, D = x.shape
    E, H, _ = ew1.shape
    O = ew2.shape[1]

    xf = x.reshape(B * F, D)
    w1t = ew1.reshape(E * H, D).T.astype(jnp.bfloat16)
    gw1t = gw1.T.astype(jnp.bfloat16)
    wcat = jnp.concatenate(
        [w1t, gw1t, jnp.zeros((D, GPAD - H), jnp.bfloat16)], axis=1)
    bcat = jnp.concatenate(
        [eb1.reshape(1, E * H), gb1.reshape(1, H),
         jnp.zeros((1, GPAD - H), jnp.float32)], axis=1)
    gw2tp = jnp.concatenate(
        [gw2.T, jnp.zeros((GPAD - H, E), jnp.float32)], axis=0)
    gb2r = gb2.reshape(1, E)
    w2 = ew2.transpose(0, 2, 1).reshape(E * H, O).astype(jnp.bfloat16)
    emat = jnp.kron(jnp.eye(E, dtype=x.dtype), jnp.ones((1, H), dtype=x.dtype))

    full = lambda *shape: pl.BlockSpec(shape, lambda b: (0,) * len(shape))
    out = pl.pallas_call(
        _moe_body,
        grid=(B,),
        in_specs=[
            pl.BlockSpec((F, D), lambda b: (b, 0)),
            full(D, FLAT + GPAD), full(1, FLAT + GPAD),
            full(GPAD, E), full(1, E),
            full(E * H, O), full(E, O),
            full(E, E * H),
        ],
        out_specs=pl.BlockSpec((1, 1, O), lambda b: (b, 0, 0)),
        out_shape=jax.ShapeDtypeStruct((B, 1, O), x.dtype),
    )(xf, wcat, bcat, gw2tp, gb2r, w2, eb2, emat)
    return out.reshape(B, 1, 1, O)
